# Initial kernel scaffold; baseline (speedup 1.0000x reference)
import jax
import jax.numpy as jnp
from jax.experimental import pallas as pl


def kernel(x, edge_index, edge_weight, W):
    raise NotImplementedError("write your pallas kernel here")


# speedup vs baseline: 5.3017x; 5.3017x over previous
import functools

import jax
import jax.numpy as jnp
from jax import lax
from jax.experimental import pallas as pl
from jax.experimental.pallas import tpu as pltpu
from jax.experimental.pallas import tpu_sc as plsc

N = 10000
NPAD = 10240
E = 320000
DIN = 128
DOUT = 128
CHUNK = 128
EDGES_PER_CORE = E // 2
NUM_CHUNKS = EDGES_PER_CORE // CHUNK
NS = 16
ROWS_PER_TILE = NPAD // NS
RB = 128
CHUNKS_PER_TILE = -(-NUM_CHUNKS // NS)


def _mm_body(x_ref, w_ref, o_ref):
    o_ref[...] = jnp.dot(x_ref[...], w_ref[...], preferred_element_type=jnp.float32)


def _matmul(x, W):
    bm = 1000
    return pl.pallas_call(
        _mm_body,
        grid=(N // bm,),
        in_specs=[
            pl.BlockSpec((bm, DIN), lambda i: (i, 0)),
            pl.BlockSpec((DIN, DOUT), lambda i: (0, 0)),
        ],
        out_specs=pl.BlockSpec((bm, DOUT), lambda i: (i, 0)),
        out_shape=jax.ShapeDtypeStruct((N, DOUT), jnp.float32),
    )(x, W)


def _combine_body(a_ref, b_ref, o_ref):
    o_ref[...] = jnp.maximum(a_ref[...] + b_ref[...], 0.0)


def _combine_relu(p0, p1):
    bm = 1000
    return pl.pallas_call(
        _combine_body,
        grid=(N // bm,),
        in_specs=[
            pl.BlockSpec((bm, DOUT), lambda i: (i, 0)),
            pl.BlockSpec((bm, DOUT), lambda i: (i, 0)),
        ],
        out_specs=pl.BlockSpec((bm, DOUT), lambda i: (i, 0)),
        out_shape=jax.ShapeDtypeStruct((N, DOUT), jnp.float32),
    )(p0, p1)


@functools.partial(
    pl.kernel,
    out_type=jax.ShapeDtypeStruct((2, NPAD, DOUT), jnp.float32),
    mesh=plsc.VectorSubcoreMesh(core_axis_name="c", subcore_axis_name="s"),
    scratch_types=[
        pltpu.VMEM((CHUNK,), jnp.int32),
        pltpu.VMEM((CHUNK,), jnp.int32),
        pltpu.VMEM((CHUNK,), jnp.float32),
        pltpu.VMEM((CHUNK, DOUT), jnp.float32),
        pltpu.VMEM_SHARED((NPAD, DOUT), jnp.float32),
        pltpu.SemaphoreType.DMA,
    ],
)
def _sc_aggregate(pre_hbm, src_hbm, dst_hbm, ew_hbm, out_hbm,
                  src_v, dst_v, ew_v, rows_v, acc, sem):
    c = lax.axis_index("c")
    s = lax.axis_index("s")
    row0 = s * ROWS_PER_TILE

    def _zero_row(r, carry):
        for j in range(DOUT // 16):
            rows_v[r, pl.ds(j * 16, 16)] = jnp.zeros((16,), jnp.float32)
        return carry

    lax.fori_loop(0, CHUNK, _zero_row, 0)
    for b in range(ROWS_PER_TILE // RB):
        pltpu.sync_copy(rows_v.at[pl.ds(0, RB)],
                        acc.at[pl.ds(row0 + b * RB, RB)])
    plsc.subcore_barrier()

    def _chunk(i, carry):
        g = s + i * NS

        @pl.when(g < NUM_CHUNKS)
        def _():
            e0 = c * EDGES_PER_CORE + g * CHUNK
            pltpu.sync_copy(src_hbm.at[pl.ds(e0, CHUNK)], src_v)
            pltpu.sync_copy(dst_hbm.at[pl.ds(e0, CHUNK)], dst_v)
            pltpu.sync_copy(ew_hbm.at[pl.ds(e0, CHUNK)], ew_v)
            pltpu.async_copy(pre_hbm.at[src_v], rows_v, sem).wait()

            def _scale(eg, carry2):
                w16 = ew_v[pl.ds(eg * 16, 16)]
                for k in range(16):
                    e = eg * 16 + k
                    wk = w16[k]
                    for j in range(DOUT // 16):
                        sl = pl.ds(j * 16, 16)
                        rows_v[e, sl] = rows_v[e, sl] * wk
                return carry2

            lax.fori_loop(0, CHUNK // 16, _scale, 0)
            pltpu.sync_copy(rows_v, acc.at[dst_v], add=True)

        return carry

    lax.fori_loop(0, CHUNKS_PER_TILE, _chunk, 0)
    plsc.subcore_barrier()

    pltpu.sync_copy(acc.at[pl.ds(row0, ROWS_PER_TILE)],
                    out_hbm.at[c, pl.ds(row0, ROWS_PER_TILE)])


def kernel(x, edge_index, edge_weight, W):
    pre = _matmul(x, W)
    partials = _sc_aggregate(pre, edge_index[0], edge_index[1], edge_weight)
    return _combine_relu(partials[0], partials[1])
